# baseline (device time: 59640 ns/iter reference)
import jax
import jax.numpy as jnp
from jax import lax
from jax.experimental import pallas as pl
from jax.experimental.pallas import tpu as pltpu

N = 4
R = 128
W = 768
XW = 512


def kernel(partial, resid, gamma):
    _, M, D = partial.shape

    def body(partial_ref, resid_ref, gamma_ref, out_ref,
             pstage, rstage, psend, precv,
             send_x, recv_x, dma_p, dma_r,
             ln_send, ln_recv, pc_send, pc_recv,
             xl_send, xl_recv, xp_send, xp_recv):
        my_x = lax.axis_index("x")
        my_y = lax.axis_index("y")
        my_z = lax.axis_index("z")
        xpeer = (1 - my_x, my_y, my_z)

        barrier_sem = pltpu.get_barrier_semaphore()
        pl.semaphore_signal(barrier_sem, inc=1, device_id=xpeer,
                            device_id_type=pl.DeviceIdType.MESH)

        @pl.when(my_y > 0)
        def _():
            pl.semaphore_signal(barrier_sem, inc=1,
                                device_id=(my_x, my_y - 1, my_z),
                                device_id_type=pl.DeviceIdType.MESH)

        @pl.when(my_y < N - 1)
        def _():
            pl.semaphore_signal(barrier_sem, inc=1,
                                device_id=(my_x, my_y + 1, my_z),
                                device_id_type=pl.DeviceIdType.MESH)

        @pl.when(my_z > 0)
        def _():
            pl.semaphore_signal(barrier_sem, inc=1,
                                device_id=(my_x, my_y, my_z - 1),
                                device_id_type=pl.DeviceIdType.MESH)

        @pl.when(my_z < N - 1)
        def _():
            pl.semaphore_signal(barrier_sem, inc=1,
                                device_id=(my_x, my_y, my_z + 1),
                                device_id_type=pl.DeviceIdType.MESH)

        c_me = N * my_y + my_z
        cp = pltpu.make_async_copy(
            partial_ref.at[0, pl.ds(c_me * R, R)], pstage, dma_p)
        cp.start()
        cr = pltpu.make_async_copy(
            resid_ref.at[pl.ds(c_me * R, R)], rstage, dma_r)
        cr.start()
        cp.wait()
        psend[...] = pstage[...].astype(jnp.bfloat16)

        n_nbrs = (1
                  + (my_y > 0).astype(jnp.int32)
                  + (my_y < N - 1).astype(jnp.int32)
                  + (my_z > 0).astype(jnp.int32)
                  + (my_z < N - 1).astype(jnp.int32))
        pl.semaphore_wait(barrier_sem, n_nbrs)

        rx = pltpu.make_async_remote_copy(
            src_ref=psend, dst_ref=precv, send_sem=send_x, recv_sem=recv_x,
            device_id=xpeer, device_id_type=pl.DeviceIdType.MESH)
        rx.start()
        rx.wait()
        cr.wait()

        y = (psend[...].astype(jnp.float32)
             + precv[...].astype(jnp.float32)
             + rstage[...])
        rms = jnp.sqrt(jnp.mean(y * y, axis=-1, keepdims=True) + 1e-6)
        o = (y / rms * gamma_ref[...][None, :]).astype(jnp.bfloat16)
        out_ref[pl.ds(c_me * R, R), :] = o

        A, B = 0, 1

        def peer(stream, axis_is_piece, d):
            if (stream == A) == (not axis_is_piece):
                return (my_x, my_y, my_z + d)
            return (my_x, my_y + d, my_z)

        def lpos(stream):
            return my_z if stream == A else my_y

        def ppos(stream):
            return my_y if stream == A else my_z

        def unit(stream, fp, lz):
            if stream == A:
                return out_ref.at[pl.ds((N * fp + lz) * R, R),
                                  pl.ds(my_x * XW, W)]
            return out_ref.at[pl.ds((N * lz + fp) * R, R),
                              pl.ds(my_x * XW + W, W)]

        def xfwd(stream, fp, lz, sem_s, sem_r):
            xok = (my_x == 0) if stream == A else (my_x == 1)
            @pl.when(xok)
            def _():
                if stream == A:
                    blk = out_ref.at[pl.ds((N * fp + lz) * R, R),
                                     pl.ds(0, XW)]
                else:
                    blk = out_ref.at[pl.ds((N * lz + fp) * R, R),
                                     pl.ds(3 * XW, XW)]
                r = pltpu.make_async_remote_copy(
                    src_ref=blk, dst_ref=blk,
                    send_sem=sem_s, recv_sem=sem_r,
                    device_id=xpeer, device_id_type=pl.DeviceIdType.MESH)
                r.start()

        def piece_send(stream, s, lz):
            pp = ppos(stream)
            @pl.when((pp - s >= 0) & (pp < N - 1))
            def _():
                r = pltpu.make_async_remote_copy(
                    src_ref=unit(stream, pp - s, lz),
                    dst_ref=unit(stream, pp - s, lz),
                    send_sem=pc_send.at[stream, 0, s, lz],
                    recv_sem=pc_recv.at[stream, 0, s, lz],
                    device_id=peer(stream, True, 1),
                    device_id_type=pl.DeviceIdType.MESH)
                r.start()

            @pl.when((pp + s <= N - 1) & (pp > 0))
            def _():
                r = pltpu.make_async_remote_copy(
                    src_ref=unit(stream, pp + s, lz),
                    dst_ref=unit(stream, pp + s, lz),
                    send_sem=pc_send.at[stream, 1, s, lz],
                    recv_sem=pc_recv.at[stream, 1, s, lz],
                    device_id=peer(stream, True, -1),
                    device_id_type=pl.DeviceIdType.MESH)
                r.start()

        def piece_wait_recv(stream, h, lz):
            pp = ppos(stream)
            @pl.when(pp >= h + 1)
            def _():
                r = pltpu.make_async_remote_copy(
                    src_ref=unit(stream, pp - 1 - h, lz),
                    dst_ref=unit(stream, pp - 1 - h, lz),
                    send_sem=pc_send.at[stream, 0, h, lz],
                    recv_sem=pc_recv.at[stream, 0, h, lz],
                    device_id=peer(stream, True, -1),
                    device_id_type=pl.DeviceIdType.MESH)
                r.wait_recv()
                xfwd(stream, pp - 1 - h, lz,
                     xp_send.at[h, lz, 0], xp_recv.at[h, lz, 0])

            @pl.when(pp + 1 + h <= N - 1)
            def _():
                r = pltpu.make_async_remote_copy(
                    src_ref=unit(stream, pp + 1 + h, lz),
                    dst_ref=unit(stream, pp + 1 + h, lz),
                    send_sem=pc_send.at[stream, 1, h, lz],
                    recv_sem=pc_recv.at[stream, 1, h, lz],
                    device_id=peer(stream, True, 1),
                    device_id_type=pl.DeviceIdType.MESH)
                r.wait_recv()
                xfwd(stream, pp + 1 + h, lz,
                     xp_send.at[h, lz, 1], xp_recv.at[h, lz, 1])

        def piece_wait_send(stream, s, lz):
            pp = ppos(stream)
            @pl.when((pp - s >= 0) & (pp < N - 1))
            def _():
                r = pltpu.make_async_remote_copy(
                    src_ref=unit(stream, pp - s, lz),
                    dst_ref=unit(stream, pp - s, lz),
                    send_sem=pc_send.at[stream, 0, s, lz],
                    recv_sem=pc_recv.at[stream, 0, s, lz],
                    device_id=peer(stream, True, 1),
                    device_id_type=pl.DeviceIdType.MESH)
                r.wait_send()

            @pl.when((pp + s <= N - 1) & (pp > 0))
            def _():
                r = pltpu.make_async_remote_copy(
                    src_ref=unit(stream, pp + s, lz),
                    dst_ref=unit(stream, pp + s, lz),
                    send_sem=pc_send.at[stream, 1, s, lz],
                    recv_sem=pc_recv.at[stream, 1, s, lz],
                    device_id=peer(stream, True, -1),
                    device_id_type=pl.DeviceIdType.MESH)
                r.wait_send()

        def line_descr(stream, s, dirn, idx):
            return pltpu.make_async_remote_copy(
                src_ref=unit(stream, ppos(stream), idx),
                dst_ref=unit(stream, ppos(stream), idx),
                send_sem=ln_send.at[stream, s, dirn],
                recv_sem=ln_recv.at[stream, s, dirn],
                device_id=peer(stream, False, 1 if dirn == 0 else -1),
                device_id_type=pl.DeviceIdType.MESH)

        def line_send(stream, s):
            lp = lpos(stream)
            @pl.when((lp >= s) & (lp < N - 1))
            def _():
                line_descr(stream, s, 0, lp - s).start()

            @pl.when((lp > 0) & (lp + s <= N - 1))
            def _():
                line_descr(stream, s, 1, lp + s).start()

        def line_wait_recv(stream, s):
            lp = lpos(stream)
            @pl.when(lp >= s + 1)
            def _():
                line_descr(stream, s, 0, lp - 1 - s).wait_recv()
                xfwd(stream, ppos(stream), lp - 1 - s,
                     xl_send.at[s, 0], xl_recv.at[s, 0])

            @pl.when(lp + 1 + s <= N - 1)
            def _():
                line_descr(stream, s, 1, lp + 1 + s).wait_recv()
                xfwd(stream, ppos(stream), lp + 1 + s,
                     xl_send.at[s, 1], xl_recv.at[s, 1])

        def line_wait_send(stream, s):
            lp = lpos(stream)
            @pl.when((lp >= s) & (lp < N - 1))
            def _():
                line_descr(stream, s, 0, lp - s).wait_send()

            @pl.when((lp > 0) & (lp + s <= N - 1))
            def _():
                line_descr(stream, s, 1, lp + s).wait_send()

        for st in (A, B):
            line_send(st, 0)
        for st in (A, B):
            piece_send(st, 0, lpos(st))

        for s in range(N - 1):
            if s > 0:
                for st in (A, B):
                    line_send(st, s)
            for st in (A, B):
                line_wait_recv(st, s)
            for st in (A, B):
                lp = lpos(st)
                @pl.when(lp - (s + 1) >= 0)
                def _(st=st, s=s, lp=lp):
                    piece_send(st, 0, lp - (s + 1))

                @pl.when(lp + (s + 1) <= N - 1)
                def _(st=st, s=s, lp=lp):
                    piece_send(st, 0, lp + (s + 1))

        for s in range(1, N - 1):
            for k in range(N):
                for sgn in ((0,) if k == 0 else (-1, 1)):
                    for st in (A, B):
                        lzt = lpos(st) + sgn * k

                        @pl.when((lzt >= 0) & (lzt <= N - 1))
                        def _(st=st, s=s, lzt=lzt):
                            piece_wait_recv(st, s - 1, lzt)
                            piece_send(st, s, lzt)

        for k in range(N):
            for sgn in ((0,) if k == 0 else (-1, 1)):
                for st in (A, B):
                    lzt = lpos(st) + sgn * k

                    @pl.when((lzt >= 0) & (lzt <= N - 1))
                    def _(st=st, lzt=lzt):
                        piece_wait_recv(st, N - 2, lzt)

        plp = jnp.where(my_x == 0, my_y, my_z)
        ppp = jnp.where(my_x == 0, my_z, my_y)
        xcol = (1 - my_x) * 3 * XW

        def xrow(fp, lz):
            return jnp.where(my_x == 0, N * lz + fp, N * fp + lz) * R

        def xwait(fp, lz, sem_s, sem_r):
            blk = out_ref.at[pl.ds(xrow(fp, lz), R), pl.ds(xcol, XW)]
            r = pltpu.make_async_remote_copy(
                src_ref=blk, dst_ref=blk, send_sem=sem_s, recv_sem=sem_r,
                device_id=xpeer, device_id_type=pl.DeviceIdType.MESH)
            r.wait_recv()

        for s in range(N - 1):
            @pl.when(plp >= s + 1)
            def _(s=s):
                xwait(ppp, plp - 1 - s, xl_send.at[s, 0], xl_recv.at[s, 0])

            @pl.when(plp + 1 + s <= N - 1)
            def _(s=s):
                xwait(ppp, plp + 1 + s, xl_send.at[s, 1], xl_recv.at[s, 1])

        for h in range(N - 1):
            for lz in range(N):
                @pl.when(ppp >= h + 1)
                def _(h=h, lz=lz):
                    xwait(ppp - 1 - h, lz,
                          xp_send.at[h, lz, 0], xp_recv.at[h, lz, 0])

                @pl.when(ppp + 1 + h <= N - 1)
                def _(h=h, lz=lz):
                    xwait(ppp + 1 + h, lz,
                          xp_send.at[h, lz, 1], xp_recv.at[h, lz, 1])

        for st in (A, B):
            for s in range(N - 1):
                line_wait_send(st, s)
        for st in (A, B):
            for s in range(N - 1):
                for lz in range(N):
                    piece_wait_send(st, s, lz)

        def xdrain(sem_s, sem_r, cond):
            @pl.when(cond)
            def _():
                blk = out_ref.at[pl.ds(0, R), pl.ds(0, XW)]
                r = pltpu.make_async_remote_copy(
                    src_ref=blk, dst_ref=blk, send_sem=sem_s, recv_sem=sem_r,
                    device_id=xpeer, device_id_type=pl.DeviceIdType.MESH)
                r.wait_send()

        mlp = jnp.where(my_x == 0, my_z, my_y)
        mpp = jnp.where(my_x == 0, my_y, my_z)
        for s in range(N - 1):
            xdrain(xl_send.at[s, 0], xl_recv.at[s, 0], mlp >= s + 1)
            xdrain(xl_send.at[s, 1], xl_recv.at[s, 1], mlp + 1 + s <= N - 1)
        for h in range(N - 1):
            for lz in range(N):
                xdrain(xp_send.at[h, lz, 0], xp_recv.at[h, lz, 0],
                       mpp >= h + 1)
                xdrain(xp_send.at[h, lz, 1], xp_recv.at[h, lz, 1],
                       mpp + 1 + h <= N - 1)

    return pl.pallas_call(
        body,
        out_shape=jax.ShapeDtypeStruct((M, D), jnp.bfloat16),
        in_specs=[
            pl.BlockSpec(memory_space=pltpu.MemorySpace.HBM),
            pl.BlockSpec(memory_space=pltpu.MemorySpace.HBM),
            pl.BlockSpec(memory_space=pltpu.MemorySpace.VMEM),
        ],
        out_specs=pl.BlockSpec(memory_space=pltpu.MemorySpace.VMEM),
        scratch_shapes=[
            pltpu.VMEM((R, D), jnp.float32),
            pltpu.VMEM((R, D), jnp.float32),
            pltpu.VMEM((R, D), jnp.bfloat16),
            pltpu.VMEM((R, D), jnp.bfloat16),
            pltpu.SemaphoreType.DMA,
            pltpu.SemaphoreType.DMA,
            pltpu.SemaphoreType.DMA,
            pltpu.SemaphoreType.DMA,
            pltpu.SemaphoreType.DMA((2, N - 1, 2)),
            pltpu.SemaphoreType.DMA((2, N - 1, 2)),
            pltpu.SemaphoreType.DMA((2, 2, N - 1, N)),
            pltpu.SemaphoreType.DMA((2, 2, N - 1, N)),
            pltpu.SemaphoreType.DMA((N - 1, 2)),
            pltpu.SemaphoreType.DMA((N - 1, 2)),
            pltpu.SemaphoreType.DMA((N - 1, N, 2)),
            pltpu.SemaphoreType.DMA((N - 1, N, 2)),
        ],
        compiler_params=pltpu.CompilerParams(collective_id=0),
    )(partial, resid, gamma)


# device time: 59277 ns/iter; 1.0061x vs baseline; 1.0061x over previous
import jax
import jax.numpy as jnp
from jax import lax
from jax.experimental import pallas as pl
from jax.experimental.pallas import tpu as pltpu

N = 4
R = 128
W = 768
XW = 512


def kernel(partial, resid, gamma):
    _, M, D = partial.shape

    def body(partial_ref, resid_ref, gamma_ref, out_ref,
             pstage, rstage, psend, precv,
             send_x, recv_x, send_x2, recv_x2, dma_p, dma_r,
             ln_send, ln_recv, pc_send, pc_recv,
             xl_send, xl_recv, xp_send, xp_recv):
        my_x = lax.axis_index("x")
        my_y = lax.axis_index("y")
        my_z = lax.axis_index("z")
        xpeer = (1 - my_x, my_y, my_z)

        barrier_sem = pltpu.get_barrier_semaphore()
        pl.semaphore_signal(barrier_sem, inc=1, device_id=xpeer,
                            device_id_type=pl.DeviceIdType.MESH)

        @pl.when(my_y > 0)
        def _():
            pl.semaphore_signal(barrier_sem, inc=1,
                                device_id=(my_x, my_y - 1, my_z),
                                device_id_type=pl.DeviceIdType.MESH)

        @pl.when(my_y < N - 1)
        def _():
            pl.semaphore_signal(barrier_sem, inc=1,
                                device_id=(my_x, my_y + 1, my_z),
                                device_id_type=pl.DeviceIdType.MESH)

        @pl.when(my_z > 0)
        def _():
            pl.semaphore_signal(barrier_sem, inc=1,
                                device_id=(my_x, my_y, my_z - 1),
                                device_id_type=pl.DeviceIdType.MESH)

        @pl.when(my_z < N - 1)
        def _():
            pl.semaphore_signal(barrier_sem, inc=1,
                                device_id=(my_x, my_y, my_z + 1),
                                device_id_type=pl.DeviceIdType.MESH)

        c_me = N * my_y + my_z
        cp = pltpu.make_async_copy(
            partial_ref.at[0, pl.ds(c_me * R, R)], pstage, dma_p)
        cp.start()
        cr = pltpu.make_async_copy(
            resid_ref.at[pl.ds(c_me * R, R)], rstage, dma_r)
        cr.start()
        cp.wait()
        psend[...] = pstage[...].astype(jnp.bfloat16)

        n_nbrs = (1
                  + (my_y > 0).astype(jnp.int32)
                  + (my_y < N - 1).astype(jnp.int32)
                  + (my_z > 0).astype(jnp.int32)
                  + (my_z < N - 1).astype(jnp.int32))
        pl.semaphore_wait(barrier_sem, n_nbrs)

        HR = R // 2
        rx1 = pltpu.make_async_remote_copy(
            src_ref=psend.at[pl.ds(0, HR)], dst_ref=precv.at[pl.ds(0, HR)],
            send_sem=send_x, recv_sem=recv_x,
            device_id=xpeer, device_id_type=pl.DeviceIdType.MESH)
        rx1.start()
        rx2 = pltpu.make_async_remote_copy(
            src_ref=psend.at[pl.ds(HR, HR)], dst_ref=precv.at[pl.ds(HR, HR)],
            send_sem=send_x2, recv_sem=recv_x2,
            device_id=xpeer, device_id_type=pl.DeviceIdType.MESH)
        rx2.start()
        cr.wait()

        def half(lo):
            yh = (psend[lo:lo + HR].astype(jnp.float32)
                  + precv[lo:lo + HR].astype(jnp.float32)
                  + rstage[lo:lo + HR])
            rms = jnp.sqrt(jnp.mean(yh * yh, axis=-1, keepdims=True) + 1e-6)
            oh = (yh / rms * gamma_ref[...][None, :]).astype(jnp.bfloat16)
            out_ref[pl.ds(c_me * R + lo, HR), :] = oh

        rx1.wait()
        half(0)
        rx2.wait()
        half(HR)

        A, B = 0, 1

        def peer(stream, axis_is_piece, d):
            if (stream == A) == (not axis_is_piece):
                return (my_x, my_y, my_z + d)
            return (my_x, my_y + d, my_z)

        def lpos(stream):
            return my_z if stream == A else my_y

        def ppos(stream):
            return my_y if stream == A else my_z

        def unit(stream, fp, lz):
            if stream == A:
                return out_ref.at[pl.ds((N * fp + lz) * R, R),
                                  pl.ds(my_x * XW, W)]
            return out_ref.at[pl.ds((N * lz + fp) * R, R),
                              pl.ds(my_x * XW + W, W)]

        def xfwd(stream, fp, lz, sem_s, sem_r):
            xok = (my_x == 0) if stream == A else (my_x == 1)
            @pl.when(xok)
            def _():
                if stream == A:
                    blk = out_ref.at[pl.ds((N * fp + lz) * R, R),
                                     pl.ds(0, XW)]
                else:
                    blk = out_ref.at[pl.ds((N * lz + fp) * R, R),
                                     pl.ds(3 * XW, XW)]
                r = pltpu.make_async_remote_copy(
                    src_ref=blk, dst_ref=blk,
                    send_sem=sem_s, recv_sem=sem_r,
                    device_id=xpeer, device_id_type=pl.DeviceIdType.MESH)
                r.start()

        def piece_send(stream, s, lz):
            pp = ppos(stream)
            @pl.when((pp - s >= 0) & (pp < N - 1))
            def _():
                r = pltpu.make_async_remote_copy(
                    src_ref=unit(stream, pp - s, lz),
                    dst_ref=unit(stream, pp - s, lz),
                    send_sem=pc_send.at[stream, 0, s, lz],
                    recv_sem=pc_recv.at[stream, 0, s, lz],
                    device_id=peer(stream, True, 1),
                    device_id_type=pl.DeviceIdType.MESH)
                r.start()

            @pl.when((pp + s <= N - 1) & (pp > 0))
            def _():
                r = pltpu.make_async_remote_copy(
                    src_ref=unit(stream, pp + s, lz),
                    dst_ref=unit(stream, pp + s, lz),
                    send_sem=pc_send.at[stream, 1, s, lz],
                    recv_sem=pc_recv.at[stream, 1, s, lz],
                    device_id=peer(stream, True, -1),
                    device_id_type=pl.DeviceIdType.MESH)
                r.start()

        def piece_wait_recv(stream, h, lz):
            pp = ppos(stream)
            @pl.when(pp >= h + 1)
            def _():
                r = pltpu.make_async_remote_copy(
                    src_ref=unit(stream, pp - 1 - h, lz),
                    dst_ref=unit(stream, pp - 1 - h, lz),
                    send_sem=pc_send.at[stream, 0, h, lz],
                    recv_sem=pc_recv.at[stream, 0, h, lz],
                    device_id=peer(stream, True, -1),
                    device_id_type=pl.DeviceIdType.MESH)
                r.wait_recv()
                xfwd(stream, pp - 1 - h, lz,
                     xp_send.at[h, lz, 0], xp_recv.at[h, lz, 0])

            @pl.when(pp + 1 + h <= N - 1)
            def _():
                r = pltpu.make_async_remote_copy(
                    src_ref=unit(stream, pp + 1 + h, lz),
                    dst_ref=unit(stream, pp + 1 + h, lz),
                    send_sem=pc_send.at[stream, 1, h, lz],
                    recv_sem=pc_recv.at[stream, 1, h, lz],
                    device_id=peer(stream, True, 1),
                    device_id_type=pl.DeviceIdType.MESH)
                r.wait_recv()
                xfwd(stream, pp + 1 + h, lz,
                     xp_send.at[h, lz, 1], xp_recv.at[h, lz, 1])

        def piece_wait_send(stream, s, lz):
            pp = ppos(stream)
            @pl.when((pp - s >= 0) & (pp < N - 1))
            def _():
                r = pltpu.make_async_remote_copy(
                    src_ref=unit(stream, pp - s, lz),
                    dst_ref=unit(stream, pp - s, lz),
                    send_sem=pc_send.at[stream, 0, s, lz],
                    recv_sem=pc_recv.at[stream, 0, s, lz],
                    device_id=peer(stream, True, 1),
                    device_id_type=pl.DeviceIdType.MESH)
                r.wait_send()

            @pl.when((pp + s <= N - 1) & (pp > 0))
            def _():
                r = pltpu.make_async_remote_copy(
                    src_ref=unit(stream, pp + s, lz),
                    dst_ref=unit(stream, pp + s, lz),
                    send_sem=pc_send.at[stream, 1, s, lz],
                    recv_sem=pc_recv.at[stream, 1, s, lz],
                    device_id=peer(stream, True, -1),
                    device_id_type=pl.DeviceIdType.MESH)
                r.wait_send()

        def line_descr(stream, s, dirn, idx):
            return pltpu.make_async_remote_copy(
                src_ref=unit(stream, ppos(stream), idx),
                dst_ref=unit(stream, ppos(stream), idx),
                send_sem=ln_send.at[stream, s, dirn],
                recv_sem=ln_recv.at[stream, s, dirn],
                device_id=peer(stream, False, 1 if dirn == 0 else -1),
                device_id_type=pl.DeviceIdType.MESH)

        def line_send(stream, s):
            lp = lpos(stream)
            @pl.when((lp >= s) & (lp < N - 1))
            def _():
                line_descr(stream, s, 0, lp - s).start()

            @pl.when((lp > 0) & (lp + s <= N - 1))
            def _():
                line_descr(stream, s, 1, lp + s).start()

        def line_wait_recv(stream, s):
            lp = lpos(stream)
            @pl.when(lp >= s + 1)
            def _():
                line_descr(stream, s, 0, lp - 1 - s).wait_recv()
                xfwd(stream, ppos(stream), lp - 1 - s,
                     xl_send.at[s, 0], xl_recv.at[s, 0])

            @pl.when(lp + 1 + s <= N - 1)
            def _():
                line_descr(stream, s, 1, lp + 1 + s).wait_recv()
                xfwd(stream, ppos(stream), lp + 1 + s,
                     xl_send.at[s, 1], xl_recv.at[s, 1])

        def line_wait_send(stream, s):
            lp = lpos(stream)
            @pl.when((lp >= s) & (lp < N - 1))
            def _():
                line_descr(stream, s, 0, lp - s).wait_send()

            @pl.when((lp > 0) & (lp + s <= N - 1))
            def _():
                line_descr(stream, s, 1, lp + s).wait_send()

        for st in (A, B):
            line_send(st, 0)
        for st in (A, B):
            piece_send(st, 0, lpos(st))

        for s in range(N - 1):
            if s > 0:
                for st in (A, B):
                    line_send(st, s)
            for st in (A, B):
                line_wait_recv(st, s)
            for st in (A, B):
                lp = lpos(st)
                @pl.when(lp - (s + 1) >= 0)
                def _(st=st, s=s, lp=lp):
                    piece_send(st, 0, lp - (s + 1))

                @pl.when(lp + (s + 1) <= N - 1)
                def _(st=st, s=s, lp=lp):
                    piece_send(st, 0, lp + (s + 1))

        for s in range(1, N - 1):
            for k in range(N):
                for sgn in ((0,) if k == 0 else (-1, 1)):
                    for st in (A, B):
                        lzt = lpos(st) + sgn * k

                        @pl.when((lzt >= 0) & (lzt <= N - 1))
                        def _(st=st, s=s, lzt=lzt):
                            piece_wait_recv(st, s - 1, lzt)
                            piece_send(st, s, lzt)

        for k in range(N):
            for sgn in ((0,) if k == 0 else (-1, 1)):
                for st in (A, B):
                    lzt = lpos(st) + sgn * k

                    @pl.when((lzt >= 0) & (lzt <= N - 1))
                    def _(st=st, lzt=lzt):
                        piece_wait_recv(st, N - 2, lzt)

        plp = jnp.where(my_x == 0, my_y, my_z)
        ppp = jnp.where(my_x == 0, my_z, my_y)
        xcol = (1 - my_x) * 3 * XW

        def xrow(fp, lz):
            return jnp.where(my_x == 0, N * lz + fp, N * fp + lz) * R

        def xwait(fp, lz, sem_s, sem_r):
            blk = out_ref.at[pl.ds(xrow(fp, lz), R), pl.ds(xcol, XW)]
            r = pltpu.make_async_remote_copy(
                src_ref=blk, dst_ref=blk, send_sem=sem_s, recv_sem=sem_r,
                device_id=xpeer, device_id_type=pl.DeviceIdType.MESH)
            r.wait_recv()

        for s in range(N - 1):
            @pl.when(plp >= s + 1)
            def _(s=s):
                xwait(ppp, plp - 1 - s, xl_send.at[s, 0], xl_recv.at[s, 0])

            @pl.when(plp + 1 + s <= N - 1)
            def _(s=s):
                xwait(ppp, plp + 1 + s, xl_send.at[s, 1], xl_recv.at[s, 1])

        for h in range(N - 1):
            for lz in range(N):
                @pl.when(ppp >= h + 1)
                def _(h=h, lz=lz):
                    xwait(ppp - 1 - h, lz,
                          xp_send.at[h, lz, 0], xp_recv.at[h, lz, 0])

                @pl.when(ppp + 1 + h <= N - 1)
                def _(h=h, lz=lz):
                    xwait(ppp + 1 + h, lz,
                          xp_send.at[h, lz, 1], xp_recv.at[h, lz, 1])

        for st in (A, B):
            for s in range(N - 1):
                line_wait_send(st, s)
        for st in (A, B):
            for s in range(N - 1):
                for lz in range(N):
                    piece_wait_send(st, s, lz)

        def xdrain(sem_s, sem_r, cond):
            @pl.when(cond)
            def _():
                blk = out_ref.at[pl.ds(0, R), pl.ds(0, XW)]
                r = pltpu.make_async_remote_copy(
                    src_ref=blk, dst_ref=blk, send_sem=sem_s, recv_sem=sem_r,
                    device_id=xpeer, device_id_type=pl.DeviceIdType.MESH)
                r.wait_send()

        mlp = jnp.where(my_x == 0, my_z, my_y)
        mpp = jnp.where(my_x == 0, my_y, my_z)
        for s in range(N - 1):
            xdrain(xl_send.at[s, 0], xl_recv.at[s, 0], mlp >= s + 1)
            xdrain(xl_send.at[s, 1], xl_recv.at[s, 1], mlp + 1 + s <= N - 1)
        for h in range(N - 1):
            for lz in range(N):
                xdrain(xp_send.at[h, lz, 0], xp_recv.at[h, lz, 0],
                       mpp >= h + 1)
                xdrain(xp_send.at[h, lz, 1], xp_recv.at[h, lz, 1],
                       mpp + 1 + h <= N - 1)

    return pl.pallas_call(
        body,
        out_shape=jax.ShapeDtypeStruct((M, D), jnp.bfloat16),
        in_specs=[
            pl.BlockSpec(memory_space=pltpu.MemorySpace.HBM),
            pl.BlockSpec(memory_space=pltpu.MemorySpace.HBM),
            pl.BlockSpec(memory_space=pltpu.MemorySpace.VMEM),
        ],
        out_specs=pl.BlockSpec(memory_space=pltpu.MemorySpace.VMEM),
        scratch_shapes=[
            pltpu.VMEM((R, D), jnp.float32),
            pltpu.VMEM((R, D), jnp.float32),
            pltpu.VMEM((R, D), jnp.bfloat16),
            pltpu.VMEM((R, D), jnp.bfloat16),
            pltpu.SemaphoreType.DMA,
            pltpu.SemaphoreType.DMA,
            pltpu.SemaphoreType.DMA,
            pltpu.SemaphoreType.DMA,
            pltpu.SemaphoreType.DMA,
            pltpu.SemaphoreType.DMA,
            pltpu.SemaphoreType.DMA((2, N - 1, 2)),
            pltpu.SemaphoreType.DMA((2, N - 1, 2)),
            pltpu.SemaphoreType.DMA((2, 2, N - 1, N)),
            pltpu.SemaphoreType.DMA((2, 2, N - 1, N)),
            pltpu.SemaphoreType.DMA((N - 1, 2)),
            pltpu.SemaphoreType.DMA((N - 1, 2)),
            pltpu.SemaphoreType.DMA((N - 1, N, 2)),
            pltpu.SemaphoreType.DMA((N - 1, N, 2)),
        ],
        compiler_params=pltpu.CompilerParams(collective_id=0),
    )(partial, resid, gamma)
